# baseline (speedup 1.0000x reference)
import functools

import jax
import jax.numpy as jnp
from jax import lax
from jax.experimental import pallas as pl
from jax.experimental.pallas import tpu as pltpu
from jax.experimental.pallas import tpu_sc as plsc

N = 10000
E_HYP = 10000
NNZ = 160000

_NC = 2
_NS = 16
_NW = _NC * _NS
_CHUNK = 128
_NCHUNKS = NNZ // _CHUNK
_NCHUNKS_PAD = 1280
_ITERS = _NCHUNKS_PAD // _NW
_ROWS_PER_TILE = 632
_N_PAD = _ROWS_PER_TILE * _NS


def _proj_kernel(x_ref, w_ref, b_ref, o_ref):
    o_ref[...] = (
        jnp.dot(x_ref[...], w_ref[...], preferred_element_type=jnp.float32)
        + b_ref[...]
    )


def _proj(x, w, b, block=2000):
    n, _ = x.shape
    d_out = w.shape[1]
    return pl.pallas_call(
        _proj_kernel,
        name="proj4",
        grid=(n // block,),
        in_specs=[
            pl.BlockSpec((block, x.shape[1]), lambda i: (i, 0)),
            pl.BlockSpec(w.shape, lambda i: (0, 0)),
            pl.BlockSpec((1, d_out), lambda i: (0, 0)),
        ],
        out_specs=pl.BlockSpec((block, d_out), lambda i: (i, 0)),
        out_shape=jax.ShapeDtypeStruct((n, d_out), jnp.float32),
    )(x, w, b.reshape(1, -1))


def _spmm_kernel(a_ref, y_ref, b_ref, w2_ref, b2_ref, o1_ref, *rest, relu,
                 bias, second, bias2, emit_a_bf16):
    t = jnp.dot(a_ref[...], y_ref[...], preferred_element_type=jnp.float32)
    if bias:
        t = t + b_ref[...]
    if relu:
        t = jnp.maximum(t, 0.0)
    o1_ref[...] = t
    rest = list(rest)
    if second:
        o2_ref = rest.pop(0)
        t2 = jnp.dot(t, w2_ref[...], preferred_element_type=jnp.float32)
        if bias2:
            t2 = t2 + b2_ref[...]
        o2_ref[...] = t2.astype(o2_ref.dtype)
    if emit_a_bf16:
        rest.pop(0)[...] = a_ref[...].astype(jnp.bfloat16)


def _spmm(adj, y, b=None, w2=None, b2=None, relu=False, block=400,
          o2_dtype=jnp.float32, emit_a_bf16=False, name="spmm"):
    n = adj.shape[0]
    d = y.shape[1]
    bias = b is not None
    second = w2 is not None
    bias2 = b2 is not None
    if b is None:
        b = jnp.zeros((d,), jnp.float32)
    if w2 is None:
        w2 = jnp.zeros((d, 1), jnp.float32)
    d2 = w2.shape[1]
    if b2 is None:
        b2 = jnp.zeros((d2,), jnp.float32)
    kern = functools.partial(_spmm_kernel, relu=relu, bias=bias,
                             second=second, bias2=bias2,
                             emit_a_bf16=emit_a_bf16)
    out_specs = [pl.BlockSpec((block, d), lambda i: (i, 0))]
    out_shape = [jax.ShapeDtypeStruct((n, d), jnp.float32)]
    if second:
        out_specs.append(pl.BlockSpec((block, d2), lambda i: (i, 0)))
        out_shape.append(jax.ShapeDtypeStruct((n, d2), o2_dtype))
    if emit_a_bf16:
        out_specs.append(pl.BlockSpec((block, adj.shape[1]),
                                      lambda i: (i, 0)))
        out_shape.append(jax.ShapeDtypeStruct((n, adj.shape[1]),
                                              jnp.bfloat16))
    outs = pl.pallas_call(
        kern,
        name=name,
        grid=(n // block,),
        in_specs=[
            pl.BlockSpec((block, adj.shape[1]), lambda i: (i, 0)),
            pl.BlockSpec(y.shape, lambda i: (0, 0)),
            pl.BlockSpec((1, d), lambda i: (0, 0)),
            pl.BlockSpec(w2.shape, lambda i: (0, 0)),
            pl.BlockSpec((1, d2), lambda i: (0, 0)),
        ],
        out_specs=out_specs,
        out_shape=out_shape,
    )(adj, y, b.reshape(1, -1), w2, b2.reshape(1, -1))
    outs = list(outs) if isinstance(outs, (list, tuple)) else [outs]
    o1 = outs.pop(0)
    o2 = outs.pop(0) if second else None
    o3 = outs.pop(0) if emit_a_bf16 else None
    return o1, o2, o3


def _att_kernel(z1_ref, z2_ref, w1_ref, b1_ref, w2_ref, pw_ref, pb_ref,
                o_ref, o2_ref, o3_ref, *, second, parts, emit_z2, swap):
    z1 = z1_ref[...]
    if parts:
        d = o_ref.shape[1]
        psum = z2_ref[0] + z2_ref[1]
        z2 = psum[:, :d] / (psum[:, d:d + 1] + 1e-16)
    else:
        z2 = z2_ref[...]
    if emit_z2:
        o3_ref[...] = z2
    if swap:
        z1, z2 = z2, z1
    w1 = w1_ref[...]
    b1 = b1_ref[...]
    w2 = w2_ref[...]
    wa = jnp.dot(jnp.tanh(jnp.dot(z1, w1, preferred_element_type=jnp.float32)
                          + b1), w2, preferred_element_type=jnp.float32)
    wb = jnp.dot(jnp.tanh(jnp.dot(z2, w1, preferred_element_type=jnp.float32)
                          + b1), w2, preferred_element_type=jnp.float32)
    m = jnp.maximum(wa, wb)
    ea = jnp.exp(wa - m)
    eb = jnp.exp(wb - m)
    inv = 1.0 / (ea + eb)
    out = (ea * inv) * z1 + (eb * inv) * z2
    o_ref[...] = out
    if second:
        o2_ref[...] = (
            jnp.dot(out, pw_ref[...], preferred_element_type=jnp.float32)
            + pb_ref[...]
        )


def _attention(z1, z2, w1, b1, w2, proj_w=None, proj_b=None, block=2000,
               emit_z2=False, swap=False, name="attention"):
    n, d = z1.shape
    h = w1.shape[1]
    parts = z2.ndim == 3
    second = proj_w is not None
    if proj_w is None:
        proj_w = jnp.zeros((d, 1), jnp.float32)
        proj_b = jnp.zeros((1,), jnp.float32)
    d2 = proj_w.shape[1]
    kern = functools.partial(_att_kernel, second=second, parts=parts,
                             emit_z2=emit_z2, swap=swap)
    if parts:
        width = z2.shape[2]
        z2_spec = pl.BlockSpec((2, block, width), lambda i: (0, i, 0))
    else:
        z2_spec = pl.BlockSpec((block, d), lambda i: (i, 0))
    o1, o2, o3 = pl.pallas_call(
        kern,
        name=name,
        grid=(n // block,),
        in_specs=[
            pl.BlockSpec((block, d), lambda i: (i, 0)),
            z2_spec,
            pl.BlockSpec((d, h), lambda i: (0, 0)),
            pl.BlockSpec((1, h), lambda i: (0, 0)),
            pl.BlockSpec((h, 1), lambda i: (0, 0)),
            pl.BlockSpec(proj_w.shape, lambda i: (0, 0)),
            pl.BlockSpec((1, d2), lambda i: (0, 0)),
        ],
        out_specs=[
            pl.BlockSpec((block, d), lambda i: (i, 0)),
            pl.BlockSpec((block, d2), lambda i: (i, 0)),
            pl.BlockSpec((block, d), lambda i: (i, 0)),
        ],
        out_shape=[
            jax.ShapeDtypeStruct((n, d), jnp.float32),
            jax.ShapeDtypeStruct((n, d2), jnp.float32),
            jax.ShapeDtypeStruct((n, d), jnp.float32),
        ],
    )(z1, z2, w1, b1.reshape(1, -1), w2, proj_w, proj_b.reshape(1, -1))
    return o1, o2, o3


@functools.lru_cache(maxsize=None)
def _make_sc_scatter(width):
    mesh = plsc.VectorSubcoreMesh(core_axis_name="c", subcore_axis_name="s")

    @functools.partial(
        pl.kernel,
        mesh=mesh,
        out_type=jax.ShapeDtypeStruct((_NC, _N_PAD, width), jnp.float32),
        compiler_params=pltpu.CompilerParams(use_tc_tiling_on_sc=False),
        scratch_types=[
            pltpu.VMEM((3, 1, _CHUNK), jnp.int32),
            pltpu.VMEM((3, 1, _CHUNK), jnp.int32),
            pltpu.VMEM((2, _CHUNK, width), jnp.float32),
            pltpu.VMEM_SHARED((_N_PAD, width), jnp.float32),
            pltpu.SemaphoreType.DMA,
            pltpu.SemaphoreType.DMA,
            pltpu.SemaphoreType.DMA,
            pltpu.SemaphoreType.DMA,
            pltpu.SemaphoreType.DMA,
        ],
    )
    def k(table_hbm, gidx_hbm, sidx_hbm, zeros_hbm, out_hbm,
          gb, sb, rows, acc, gsem0, gsem1, isem0, isem1, isem2):
        c = lax.axis_index("c")
        s = lax.axis_index("s")
        w = s * _NC + c
        gsems = (gsem0, gsem1)
        isems = (isem0, isem1, isem2)

        pltpu.sync_copy(zeros_hbm,
                        acc.at[pl.ds(s * _ROWS_PER_TILE, _ROWS_PER_TILE)])

        def idx_req(j):
            cid = w * _ITERS + j
            sl = j % 3
            return (pltpu.async_copy(gidx_hbm.at[cid], gb.at[sl], isems[sl]),
                    pltpu.async_copy(sidx_hbm.at[cid], sb.at[sl], isems[sl]))

        def gather(j):
            return pltpu.async_copy(table_hbm.at[gb.at[j % 3, 0]],
                                    rows.at[j % 2], gsems[j % 2])

        idesc = [idx_req(0), idx_req(1), idx_req(2)]
        plsc.subcore_barrier()

        for d in idesc[0]:
            d.wait()
        gdesc = [gather(0), None]
        for d in idesc[1]:
            d.wait()
        gdesc[1] = gather(1)
        for j in range(_ITERS):
            gdesc[j % 2].wait()
            pltpu.sync_copy(rows.at[j % 2], acc.at[sb.at[j % 3, 0]], add=True)
            if j + 3 < _ITERS:
                idesc[j % 3] = idx_req(j + 3)
            if j + 2 < _ITERS:
                for d in idesc[(j + 2) % 3]:
                    d.wait()
                gdesc[j % 2] = gather(j + 2)

        plsc.subcore_barrier()
        pltpu.sync_copy(acc.at[pl.ds(s * _ROWS_PER_TILE, _ROWS_PER_TILE)],
                        out_hbm.at[c, pl.ds(s * _ROWS_PER_TILE,
                                            _ROWS_PER_TILE)])

    return k


def _pad_idx3(idx, fill):
    idx2 = idx.reshape(_NCHUNKS, _CHUNK)
    pad = jnp.full((_NCHUNKS_PAD - _NCHUNKS, _CHUNK), fill, jnp.int32)
    full = jnp.concatenate([idx2, pad], axis=0)
    perm = full.reshape(_ITERS, _NW, _CHUNK).transpose(1, 0, 2)
    return perm.reshape(_NCHUNKS_PAD, 1, _CHUNK)


def _sc_segment_scatter(table, gidx3, sidx3, zeros):
    return _make_sc_scatter(table.shape[1])(table, gidx3, sidx3, zeros)


def _edge_combine_kernel(parts_ref, ae_ref, dep_ref, zg_ref, *, width):
    c = width - 16
    ssum = parts_ref[0] + parts_ref[1]
    cnt = ssum[:, c:c + 1]
    xe = ssum[:, :c] / jnp.maximum(cnt, 1.0)
    alpha = jnp.sum(xe * ae_ref[...], axis=1, keepdims=True)
    a = jnp.where(alpha >= 0, alpha, 0.2 * alpha) + dep_ref[0, 0]
    g = jnp.exp(a - jnp.max(a))
    zg_ref[...] = jnp.concatenate(
        [xe * g, jnp.broadcast_to(g, (a.shape[0], 16))], axis=1)


def _edge_combine(parts, att_e, dep, name="edge_combine"):
    width = parts.shape[2]
    kern = functools.partial(_edge_combine_kernel, width=width)
    return pl.pallas_call(
        kern,
        name=name,
        grid=(1,),
        in_specs=[
            pl.BlockSpec((2, E_HYP, width), lambda i: (0, 0, 0)),
            pl.BlockSpec((1, width - 16), lambda i: (0, 0)),
            pl.BlockSpec((1, 1), lambda i: (0, 0)),
        ],
        out_specs=pl.BlockSpec((E_HYP, width), lambda i: (0, 0)),
        out_shape=jax.ShapeDtypeStruct((E_HYP, width), jnp.float32),
    )(parts, att_e.reshape(1, -1), dep.reshape(1, 1))


def kernel(X_gene, H_gene, vertex_spa, edges_spa, G_gene, G_spatial, params):
    p = params
    d_hid = p['hgnn1_W'].shape[1]
    d_out = p['hgnn2_W'].shape[1]

    vertex_g = _pad_idx3(vertex_spa, 0)
    vertex_s = _pad_idx3(vertex_spa, N)
    edges_g = _pad_idx3(edges_spa, 0)
    edges_s = _pad_idx3(edges_spa, N)
    zeros_hid = jnp.zeros((_ROWS_PER_TILE, d_hid + 16), jnp.float32)
    zeros_out = jnp.zeros((_ROWS_PER_TILE, d_out + 16), jnp.float32)

    w_cat = jnp.concatenate(
        [p['hgnn1_W'], p['gcn13_W'], p['gcn14_W'], p['gat1_W'],
         jnp.zeros((d_hid, 16), jnp.float32)], axis=1)
    b_cat = jnp.concatenate(
        [p['hgnn1_b'], jnp.zeros((3 * d_hid,), jnp.float32),
         jnp.ones((16,), jnp.float32)])
    y = _proj(X_gene, w_cat, b_cat)
    y_h1 = y[:, :d_hid]
    y_13 = y[:, d_hid:2 * d_hid]
    y_14 = y[:, 2 * d_hid:3 * d_hid]
    y_gat1 = y[:, 3 * d_hid:]

    eparts1 = _sc_segment_scatter(y_gat1, vertex_g, edges_s, zeros_hid)

    w_gat2 = jnp.concatenate(
        [p['gat2_W'], jnp.zeros((d_hid, 16), jnp.float32)], axis=1)
    b_gat2 = jnp.concatenate(
        [jnp.zeros((d_out,), jnp.float32), jnp.ones((16,), jnp.float32)])
    H1, y_gat2, _ = _spmm(H_gene, y_h1, w2=w_gat2, b2=b_gat2, name="spmm_H1")
    h1, y_h2, g_bf16 = _spmm(G_gene, y_13, b=p['gcn13_b'], w2=p['gcn23_W'],
                             relu=True, o2_dtype=jnp.bfloat16,
                             emit_a_bf16=True, name="spmm_h1")
    s1, y_s2, _ = _spmm(G_spatial, y_14, b=p['gcn14_b'], w2=p['gcn24_W'],
                        relu=True, name="spmm_s1")

    eparts2 = _sc_segment_scatter(y_gat2, vertex_g, edges_s, zeros_out)

    dep_h = h1[0, 0] - h1[0, 0]
    dep_s = s1[0, 0] - s1[0, 0]
    zg1 = _edge_combine(eparts1, p['gat1_att_e'].reshape(-1), dep_h,
                        name="edge_combine1")
    nparts1 = _sc_segment_scatter(zg1, edges_g, vertex_s, zeros_hid)

    zg2 = _edge_combine(eparts2, p['gat2_att_e'].reshape(-1), dep_s,
                        name="edge_combine2")
    nparts2 = _sc_segment_scatter(zg2, edges_g, vertex_s, zeros_out)

    H3, y_out1, _ = _attention(H1, nparts1, p['att_W1'], p['att_b1'],
                               p['att_w2'], proj_w=p['hgnn2_W'],
                               proj_b=p['hgnn2_b'], name="attention1")

    out1, _, _ = _spmm(H_gene, y_out1, name="spmm_out1")
    h2, y_de, _ = _spmm(g_bf16, y_h2, b=p['gcn23_b'], w2=p['dec_W'],
                        o2_dtype=jnp.bfloat16, name="spmm_h2")
    s2, _, _ = _spmm(G_spatial, y_s2, b=p['gcn24_b'], name="spmm_s2")

    de_X, _, _ = _spmm(g_bf16, y_de, b=p['dec_b'], name="spmm_deX")

    out_atten, _, out2 = _attention(out1, nparts2, p['att1_W1'],
                                    p['att1_b1'], p['att1_w2'],
                                    emit_z2=True, swap=True,
                                    name="attention2")

    return (h2, s2, out_atten, de_X, out1, out2)


# speedup vs baseline: 1.1358x; 1.1358x over previous
import functools

import jax
import jax.numpy as jnp
from jax import lax
from jax.experimental import pallas as pl
from jax.experimental.pallas import tpu as pltpu
from jax.experimental.pallas import tpu_sc as plsc

N = 10000
E_HYP = 10000
NNZ = 160000

_NC = 2
_NS = 16
_NW = _NC * _NS
_CHUNK = 128
_NCHUNKS = NNZ // _CHUNK
_NCHUNKS_PAD = 1280
_ITERS = _NCHUNKS_PAD // _NW
_ROWS_PER_TILE = 632
_N_PAD = _ROWS_PER_TILE * _NS


def _proj_kernel(x_ref, w_ref, b_ref, o_ref):
    o_ref[...] = (
        jnp.dot(x_ref[...], w_ref[...], preferred_element_type=jnp.float32)
        + b_ref[...]
    )


def _proj(x, w, b, block=2000):
    n, _ = x.shape
    d_out = w.shape[1]
    return pl.pallas_call(
        _proj_kernel,
        name="proj4",
        grid=(n // block,),
        in_specs=[
            pl.BlockSpec((block, x.shape[1]), lambda i: (i, 0)),
            pl.BlockSpec(w.shape, lambda i: (0, 0)),
            pl.BlockSpec((1, d_out), lambda i: (0, 0)),
        ],
        out_specs=pl.BlockSpec((block, d_out), lambda i: (i, 0)),
        out_shape=jax.ShapeDtypeStruct((n, d_out), jnp.float32),
    )(x, w, b.reshape(1, -1))


def _spmm_kernel(a_ref, y_ref, b_ref, w2_ref, b2_ref, o1_ref, *rest, relu,
                 bias, second, bias2, emit_a_bf16):
    t = jnp.dot(a_ref[...], y_ref[...], preferred_element_type=jnp.float32)
    if bias:
        t = t + b_ref[...]
    if relu:
        t = jnp.maximum(t, 0.0)
    o1_ref[...] = t
    rest = list(rest)
    if second:
        o2_ref = rest.pop(0)
        t2 = jnp.dot(t, w2_ref[...], preferred_element_type=jnp.float32)
        if bias2:
            t2 = t2 + b2_ref[...]
        o2_ref[...] = t2.astype(o2_ref.dtype)
    if emit_a_bf16:
        rest.pop(0)[...] = a_ref[...].astype(jnp.bfloat16)


def _spmm(adj, y, b=None, w2=None, b2=None, relu=False, block=400,
          o2_dtype=jnp.float32, emit_a_bf16=False, name="spmm"):
    n = adj.shape[0]
    d = y.shape[1]
    bias = b is not None
    second = w2 is not None
    bias2 = b2 is not None
    if b is None:
        b = jnp.zeros((d,), jnp.float32)
    if w2 is None:
        w2 = jnp.zeros((d, 1), jnp.float32)
    d2 = w2.shape[1]
    if b2 is None:
        b2 = jnp.zeros((d2,), jnp.float32)
    kern = functools.partial(_spmm_kernel, relu=relu, bias=bias,
                             second=second, bias2=bias2,
                             emit_a_bf16=emit_a_bf16)
    out_specs = [pl.BlockSpec((block, d), lambda i: (i, 0))]
    out_shape = [jax.ShapeDtypeStruct((n, d), jnp.float32)]
    if second:
        out_specs.append(pl.BlockSpec((block, d2), lambda i: (i, 0)))
        out_shape.append(jax.ShapeDtypeStruct((n, d2), o2_dtype))
    if emit_a_bf16:
        out_specs.append(pl.BlockSpec((block, adj.shape[1]),
                                      lambda i: (i, 0)))
        out_shape.append(jax.ShapeDtypeStruct((n, adj.shape[1]),
                                              jnp.bfloat16))
    outs = pl.pallas_call(
        kern,
        name=name,
        grid=(n // block,),
        in_specs=[
            pl.BlockSpec((block, adj.shape[1]), lambda i: (i, 0)),
            pl.BlockSpec(y.shape, lambda i: (0, 0)),
            pl.BlockSpec((1, d), lambda i: (0, 0)),
            pl.BlockSpec(w2.shape, lambda i: (0, 0)),
            pl.BlockSpec((1, d2), lambda i: (0, 0)),
        ],
        out_specs=out_specs,
        out_shape=out_shape,
    )(adj, y, b.reshape(1, -1), w2, b2.reshape(1, -1))
    outs = list(outs) if isinstance(outs, (list, tuple)) else [outs]
    o1 = outs.pop(0)
    o2 = outs.pop(0) if second else None
    o3 = outs.pop(0) if emit_a_bf16 else None
    return o1, o2, o3


def _att_kernel(z1_ref, z2_ref, w1_ref, b1_ref, w2_ref, pw_ref, pb_ref,
                o_ref, o2_ref, o3_ref, *, second, parts, emit_z2, swap):
    z1 = z1_ref[...]
    if parts:
        d = o_ref.shape[1]
        psum = z2_ref[0] + z2_ref[1]
        z2 = psum[:, :d] / (psum[:, d:d + 1] + 1e-16)
    else:
        z2 = z2_ref[...]
    if emit_z2:
        o3_ref[...] = z2
    if swap:
        z1, z2 = z2, z1
    w1 = w1_ref[...]
    b1 = b1_ref[...]
    w2 = w2_ref[...]
    wa = jnp.dot(jnp.tanh(jnp.dot(z1, w1, preferred_element_type=jnp.float32)
                          + b1), w2, preferred_element_type=jnp.float32)
    wb = jnp.dot(jnp.tanh(jnp.dot(z2, w1, preferred_element_type=jnp.float32)
                          + b1), w2, preferred_element_type=jnp.float32)
    m = jnp.maximum(wa, wb)
    ea = jnp.exp(wa - m)
    eb = jnp.exp(wb - m)
    inv = 1.0 / (ea + eb)
    out = (ea * inv) * z1 + (eb * inv) * z2
    o_ref[...] = out
    if second:
        o2_ref[...] = (
            jnp.dot(out, pw_ref[...], preferred_element_type=jnp.float32)
            + pb_ref[...]
        )


def _attention(z1, z2, w1, b1, w2, proj_w=None, proj_b=None, block=2000,
               emit_z2=False, swap=False, name="attention"):
    n, d = z1.shape
    h = w1.shape[1]
    parts = z2.ndim == 3
    second = proj_w is not None
    if proj_w is None:
        proj_w = jnp.zeros((d, 1), jnp.float32)
        proj_b = jnp.zeros((1,), jnp.float32)
    d2 = proj_w.shape[1]
    kern = functools.partial(_att_kernel, second=second, parts=parts,
                             emit_z2=emit_z2, swap=swap)
    if parts:
        width = z2.shape[2]
        z2_spec = pl.BlockSpec((2, block, width), lambda i: (0, i, 0))
    else:
        z2_spec = pl.BlockSpec((block, d), lambda i: (i, 0))
    o1, o2, o3 = pl.pallas_call(
        kern,
        name=name,
        grid=(n // block,),
        in_specs=[
            pl.BlockSpec((block, d), lambda i: (i, 0)),
            z2_spec,
            pl.BlockSpec((d, h), lambda i: (0, 0)),
            pl.BlockSpec((1, h), lambda i: (0, 0)),
            pl.BlockSpec((h, 1), lambda i: (0, 0)),
            pl.BlockSpec(proj_w.shape, lambda i: (0, 0)),
            pl.BlockSpec((1, d2), lambda i: (0, 0)),
        ],
        out_specs=[
            pl.BlockSpec((block, d), lambda i: (i, 0)),
            pl.BlockSpec((block, d2), lambda i: (i, 0)),
            pl.BlockSpec((block, d), lambda i: (i, 0)),
        ],
        out_shape=[
            jax.ShapeDtypeStruct((n, d), jnp.float32),
            jax.ShapeDtypeStruct((n, d2), jnp.float32),
            jax.ShapeDtypeStruct((n, d), jnp.float32),
        ],
    )(z1, z2, w1, b1.reshape(1, -1), w2, proj_w, proj_b.reshape(1, -1))
    return o1, o2, o3


@functools.lru_cache(maxsize=None)
def _make_sc_scatter(width):
    mesh = plsc.VectorSubcoreMesh(core_axis_name="c", subcore_axis_name="s")
    _spmem_words = (width * (N + _N_PAD)
                    + 16 * (2 * _CHUNK * width + 6 * _CHUNK))
    spmem_table = _spmem_words <= 2000000

    scratch_types = [
        pltpu.VMEM((3, 1, _CHUNK), jnp.int32),
        pltpu.VMEM((3, 1, _CHUNK), jnp.int32),
        pltpu.VMEM((2, _CHUNK, width), jnp.float32),
        pltpu.VMEM_SHARED((_N_PAD, width), jnp.float32),
        pltpu.SemaphoreType.DMA,
        pltpu.SemaphoreType.DMA,
        pltpu.SemaphoreType.DMA,
        pltpu.SemaphoreType.DMA,
        pltpu.SemaphoreType.DMA,
    ]
    if spmem_table:
        scratch_types.append(pltpu.VMEM_SHARED((N, width), jnp.float32))

    @functools.partial(
        pl.kernel,
        mesh=mesh,
        out_type=jax.ShapeDtypeStruct((_NC, _N_PAD, width), jnp.float32),
        compiler_params=pltpu.CompilerParams(use_tc_tiling_on_sc=False),
        scratch_types=scratch_types,
    )
    def k(table_hbm, gidx_hbm, sidx_hbm, zeros_hbm, out_hbm,
          gb, sb, rows, acc, gsem0, gsem1, isem0, isem1, isem2, *tsp_opt):
        c = lax.axis_index("c")
        s = lax.axis_index("s")
        w = s * _NC + c
        gsems = (gsem0, gsem1)
        isems = (isem0, isem1, isem2)

        pltpu.sync_copy(zeros_hbm,
                        acc.at[pl.ds(s * _ROWS_PER_TILE, _ROWS_PER_TILE)])

        if spmem_table:
            table = tsp_opt[0]
            nfull = N - 15 * _ROWS_PER_TILE

            @pl.when(s < 15)
            def _():
                pltpu.sync_copy(
                    table_hbm.at[pl.ds(s * _ROWS_PER_TILE, _ROWS_PER_TILE)],
                    table.at[pl.ds(s * _ROWS_PER_TILE, _ROWS_PER_TILE)])

            @pl.when(s == 15)
            def _():
                pltpu.sync_copy(
                    table_hbm.at[pl.ds(15 * _ROWS_PER_TILE, nfull)],
                    table.at[pl.ds(15 * _ROWS_PER_TILE, nfull)])
        else:
            table = table_hbm

        def idx_req(j):
            cid = w * _ITERS + j
            sl = j % 3
            return (pltpu.async_copy(gidx_hbm.at[cid], gb.at[sl], isems[sl]),
                    pltpu.async_copy(sidx_hbm.at[cid], sb.at[sl], isems[sl]))

        def gather(j):
            return pltpu.async_copy(table.at[gb.at[j % 3, 0]],
                                    rows.at[j % 2], gsems[j % 2])

        idesc = [idx_req(0), idx_req(1), idx_req(2)]
        plsc.subcore_barrier()

        for d in idesc[0]:
            d.wait()
        gdesc = [gather(0), None]
        for d in idesc[1]:
            d.wait()
        gdesc[1] = gather(1)
        for j in range(_ITERS):
            gdesc[j % 2].wait()
            pltpu.sync_copy(rows.at[j % 2], acc.at[sb.at[j % 3, 0]], add=True)
            if j + 3 < _ITERS:
                idesc[j % 3] = idx_req(j + 3)
            if j + 2 < _ITERS:
                for d in idesc[(j + 2) % 3]:
                    d.wait()
                gdesc[j % 2] = gather(j + 2)

        plsc.subcore_barrier()
        pltpu.sync_copy(acc.at[pl.ds(s * _ROWS_PER_TILE, _ROWS_PER_TILE)],
                        out_hbm.at[c, pl.ds(s * _ROWS_PER_TILE,
                                            _ROWS_PER_TILE)])

    return k


def _pad_idx3(idx, fill):
    idx2 = idx.reshape(_NCHUNKS, _CHUNK)
    pad = jnp.full((_NCHUNKS_PAD - _NCHUNKS, _CHUNK), fill, jnp.int32)
    full = jnp.concatenate([idx2, pad], axis=0)
    perm = full.reshape(_ITERS, _NW, _CHUNK).transpose(1, 0, 2)
    return perm.reshape(_NCHUNKS_PAD, 1, _CHUNK)


def _sc_segment_scatter(table, gidx3, sidx3, zeros):
    return _make_sc_scatter(table.shape[1])(table, gidx3, sidx3, zeros)


def _edge_combine_kernel(parts_ref, ae_ref, dep_ref, zg_ref, *, width):
    c = width - 16
    ssum = parts_ref[0] + parts_ref[1]
    cnt = ssum[:, c:c + 1]
    xe = ssum[:, :c] / jnp.maximum(cnt, 1.0)
    alpha = jnp.sum(xe * ae_ref[...], axis=1, keepdims=True)
    a = jnp.where(alpha >= 0, alpha, 0.2 * alpha) + dep_ref[0, 0]
    g = jnp.exp(a - jnp.max(a))
    zg_ref[...] = jnp.concatenate(
        [xe * g, jnp.broadcast_to(g, (a.shape[0], 16))], axis=1)


def _edge_combine(parts, att_e, dep, name="edge_combine"):
    width = parts.shape[2]
    kern = functools.partial(_edge_combine_kernel, width=width)
    return pl.pallas_call(
        kern,
        name=name,
        grid=(1,),
        in_specs=[
            pl.BlockSpec((2, E_HYP, width), lambda i: (0, 0, 0)),
            pl.BlockSpec((1, width - 16), lambda i: (0, 0)),
            pl.BlockSpec((1, 1), lambda i: (0, 0)),
        ],
        out_specs=pl.BlockSpec((E_HYP, width), lambda i: (0, 0)),
        out_shape=jax.ShapeDtypeStruct((E_HYP, width), jnp.float32),
    )(parts, att_e.reshape(1, -1), dep.reshape(1, 1))


def kernel(X_gene, H_gene, vertex_spa, edges_spa, G_gene, G_spatial, params):
    p = params
    d_hid = p['hgnn1_W'].shape[1]
    d_out = p['hgnn2_W'].shape[1]

    vertex_g = _pad_idx3(vertex_spa, 0)
    vertex_s = _pad_idx3(vertex_spa, N)
    edges_g = _pad_idx3(edges_spa, 0)
    edges_s = _pad_idx3(edges_spa, N)
    zeros_hid = jnp.zeros((_ROWS_PER_TILE, d_hid + 16), jnp.float32)
    zeros_out = jnp.zeros((_ROWS_PER_TILE, d_out + 16), jnp.float32)

    w_cat = jnp.concatenate(
        [p['hgnn1_W'], p['gcn13_W'], p['gcn14_W'], p['gat1_W'],
         jnp.zeros((d_hid, 16), jnp.float32)], axis=1)
    b_cat = jnp.concatenate(
        [p['hgnn1_b'], jnp.zeros((3 * d_hid,), jnp.float32),
         jnp.ones((16,), jnp.float32)])
    y = _proj(X_gene, w_cat, b_cat)
    y_h1 = y[:, :d_hid]
    y_13 = y[:, d_hid:2 * d_hid]
    y_14 = y[:, 2 * d_hid:3 * d_hid]
    y_gat1 = y[:, 3 * d_hid:]

    eparts1 = _sc_segment_scatter(y_gat1, vertex_g, edges_s, zeros_hid)

    w_gat2 = jnp.concatenate(
        [p['gat2_W'], jnp.zeros((d_hid, 16), jnp.float32)], axis=1)
    b_gat2 = jnp.concatenate(
        [jnp.zeros((d_out,), jnp.float32), jnp.ones((16,), jnp.float32)])
    H1, y_gat2, _ = _spmm(H_gene, y_h1, w2=w_gat2, b2=b_gat2, name="spmm_H1")
    h1, y_h2, g_bf16 = _spmm(G_gene, y_13, b=p['gcn13_b'], w2=p['gcn23_W'],
                             relu=True, o2_dtype=jnp.bfloat16,
                             emit_a_bf16=True, name="spmm_h1")
    s1, y_s2, _ = _spmm(G_spatial, y_14, b=p['gcn14_b'], w2=p['gcn24_W'],
                        relu=True, name="spmm_s1")

    eparts2 = _sc_segment_scatter(y_gat2, vertex_g, edges_s, zeros_out)

    dep_h = h1[0, 0] - h1[0, 0]
    dep_s = s1[0, 0] - s1[0, 0]
    zg1 = _edge_combine(eparts1, p['gat1_att_e'].reshape(-1), dep_h,
                        name="edge_combine1")
    nparts1 = _sc_segment_scatter(zg1, edges_g, vertex_s, zeros_hid)

    zg2 = _edge_combine(eparts2, p['gat2_att_e'].reshape(-1), dep_s,
                        name="edge_combine2")
    nparts2 = _sc_segment_scatter(zg2, edges_g, vertex_s, zeros_out)

    H3, y_out1, _ = _attention(H1, nparts1, p['att_W1'], p['att_b1'],
                               p['att_w2'], proj_w=p['hgnn2_W'],
                               proj_b=p['hgnn2_b'], name="attention1")

    out1, _, _ = _spmm(H_gene, y_out1, name="spmm_out1")
    h2, y_de, _ = _spmm(g_bf16, y_h2, b=p['gcn23_b'], w2=p['dec_W'],
                        o2_dtype=jnp.bfloat16, name="spmm_h2")
    s2, _, _ = _spmm(G_spatial, y_s2, b=p['gcn24_b'], name="spmm_s2")

    de_X, _, _ = _spmm(g_bf16, y_de, b=p['dec_b'], name="spmm_deX")

    out_atten, _, out2 = _attention(out1, nparts2, p['att1_W1'],
                                    p['att1_b1'], p['att1_w2'],
                                    emit_z2=True, swap=True,
                                    name="attention2")

    return (h2, s2, out_atten, de_X, out1, out2)
